# baseline (device time: 68482 ns/iter reference)
import jax
import jax.numpy as jnp
import numpy as np
from jax import lax
from jax.experimental import pallas as pl
from jax.experimental.pallas import tpu as pltpu

N_DEV = 8
B = 2
SQ_LOCAL = 128
D = 512
HQ = 4
DH = 64
HD = HQ * DH


def kernel(x, Wq, Wk, Wv, Wo):
    def body(x_ref, wq_ref, wk_ref, wv_ref, wo_ref, out_ref,
             kfull, vfull, send_sems, recv_sems):
        my = lax.axis_index("i")
        left = lax.rem(my - 1 + N_DEV, N_DEV)
        right = lax.rem(my + 1, N_DEV)

        barrier_sem = pltpu.get_barrier_semaphore()
        for nbr in (left, right):
            pl.semaphore_signal(
                barrier_sem, inc=1,
                device_id=(nbr,), device_id_type=pl.DeviceIdType.MESH,
            )
        pl.semaphore_wait(barrier_sem, 2)

        f32 = jnp.float32
        lane = lax.broadcasted_iota(jnp.int32, (SQ_LOCAL, HD), 1)
        d_in_head = lax.rem(lane, DH)
        pair = (d_in_head // 2) * 2
        freq = jnp.exp(pair.astype(f32) * f32(-np.log(10000.0) / DH))
        pos = (my * SQ_LOCAL
               + lax.broadcasted_iota(jnp.int32, (SQ_LOCAL, HD), 0)).astype(f32)
        ang = pos * freq
        cos = jnp.cos(ang)
        sin = jnp.sin(ang)

        kk = lax.broadcasted_iota(jnp.int32, (HD, HD), 0)
        jj = lax.broadcasted_iota(jnp.int32, (HD, HD), 1)
        rot_pos = ((lax.rem(jj, 2) == 1) & (kk == jj - 1)).astype(f32)
        rot_neg = ((lax.rem(jj, 2) == 0) & (kk == jj + 1)).astype(f32)
        rmat = rot_pos - rot_neg

        qs = []
        for b in range(B):
            xb = x_ref[b]
            q = jnp.dot(xb, wq_ref[...], preferred_element_type=f32)
            k = jnp.dot(xb, wk_ref[...], preferred_element_type=f32)
            v = jnp.dot(xb, wv_ref[...], preferred_element_type=f32)
            q = q * cos + jnp.dot(q, rmat, preferred_element_type=f32) * sin
            k = k * cos + jnp.dot(k, rmat, preferred_element_type=f32) * sin
            qs.append(q)
            kfull[my, b] = k
            vfull[my, b] = v

        for h in range(N_DEV - 1):
            origin = lax.rem(my - h + N_DEV, N_DEV)
            rdmas = []
            for t, ref in enumerate((kfull, vfull)):
                rdma = pltpu.make_async_remote_copy(
                    src_ref=ref.at[origin],
                    dst_ref=ref.at[origin],
                    send_sem=send_sems.at[h, t],
                    recv_sem=recv_sems.at[h, t],
                    device_id=(right,),
                    device_id_type=pl.DeviceIdType.MESH,
                )
                rdma.start()
                rdmas.append(rdma)
            for rdma in rdmas:
                rdma.wait()

        for b in range(B):
            q = qs[b]
            ctx_heads = []
            for hh in range(HQ):
                qh = q[:, hh * DH:(hh + 1) * DH]
                parts = []
                for o in range(N_DEV):
                    ko = kfull[o, b][:, hh * DH:(hh + 1) * DH]
                    parts.append(lax.dot_general(
                        qh, ko, (((1,), (1,)), ((), ())),
                        preferred_element_type=f32))
                s = jnp.concatenate(parts, axis=1) * f32(0.125)
                m = jnp.max(s, axis=1, keepdims=True)
                w = jnp.exp(s - m)
                w = w / jnp.sum(w, axis=1, keepdims=True)
                ctx = jnp.zeros((SQ_LOCAL, DH), f32)
                for o in range(N_DEV):
                    vo = vfull[o, b][:, hh * DH:(hh + 1) * DH]
                    wo_blk = w[:, o * SQ_LOCAL:(o + 1) * SQ_LOCAL]
                    ctx = ctx + jnp.dot(wo_blk, vo, preferred_element_type=f32)
                ctx_heads.append(ctx)
            ctx_b = jnp.concatenate(ctx_heads, axis=1)
            out_ref[b] = jnp.dot(ctx_b, wo_ref[...], preferred_element_type=f32)

    return pl.pallas_call(
        body,
        out_shape=jax.ShapeDtypeStruct((B, SQ_LOCAL, D), jnp.float32),
        in_specs=[pl.BlockSpec(memory_space=pltpu.VMEM)] * 5,
        out_specs=pl.BlockSpec(memory_space=pltpu.VMEM),
        scratch_shapes=[
            pltpu.VMEM((N_DEV, B, SQ_LOCAL, HD), jnp.float32),
            pltpu.VMEM((N_DEV, B, SQ_LOCAL, HD), jnp.float32),
            pltpu.SemaphoreType.DMA((N_DEV - 1, 2)),
            pltpu.SemaphoreType.DMA((N_DEV - 1, 2)),
        ],
        compiler_params=pltpu.CompilerParams(collective_id=0),
    )(x, Wq, Wk, Wv, Wo)


# device time: 48070 ns/iter; 1.4246x vs baseline; 1.4246x over previous
import jax
import jax.numpy as jnp
import numpy as np
from jax import lax
from jax.experimental import pallas as pl
from jax.experimental.pallas import tpu as pltpu

N_DEV = 8
B = 2
SQ_LOCAL = 128
D = 512
HQ = 4
DH = 64
HD = HQ * DH


def kernel(x, Wq, Wk, Wv, Wo):
    def body(x_ref, wq_ref, wk_ref, wv_ref, wo_ref, out_ref,
             kvfull, send_sems, recv_sems):
        my = lax.axis_index("i")

        barrier_sem = pltpu.get_barrier_semaphore()
        for d in range(1, N_DEV):
            pl.semaphore_signal(
                barrier_sem, inc=1,
                device_id=(lax.rem(my + d, N_DEV),),
                device_id_type=pl.DeviceIdType.MESH,
            )
        pl.semaphore_wait(barrier_sem, N_DEV - 1)

        f32 = jnp.float32
        lane = lax.broadcasted_iota(jnp.int32, (SQ_LOCAL, HD), 1)
        d_in_head = lax.rem(lane, DH)
        pair = (d_in_head // 2) * 2
        freq = jnp.exp(pair.astype(f32) * f32(-np.log(10000.0) / DH))
        pos = (my * SQ_LOCAL
               + lax.broadcasted_iota(jnp.int32, (SQ_LOCAL, HD), 0)).astype(f32)
        ang = pos * freq
        cos = jnp.cos(ang)
        sin = jnp.sin(ang)

        kk = lax.broadcasted_iota(jnp.int32, (HD, HD), 0)
        jj = lax.broadcasted_iota(jnp.int32, (HD, HD), 1)
        rot_pos = ((lax.rem(jj, 2) == 1) & (kk == jj - 1)).astype(f32)
        rot_neg = ((lax.rem(jj, 2) == 0) & (kk == jj + 1)).astype(f32)
        rmat = rot_pos - rot_neg

        qs = []
        for b in range(B):
            xb = x_ref[b]
            q = jnp.dot(xb, wq_ref[...], preferred_element_type=f32)
            k = jnp.dot(xb, wk_ref[...], preferred_element_type=f32)
            v = jnp.dot(xb, wv_ref[...], preferred_element_type=f32)
            q = q * cos + jnp.dot(q, rmat, preferred_element_type=f32) * sin
            k = k * cos + jnp.dot(k, rmat, preferred_element_type=f32) * sin
            qs.append(q)
            kvfull[my, b, 0:SQ_LOCAL] = k
            kvfull[my, b, SQ_LOCAL:2 * SQ_LOCAL] = v

        sends = []
        for d in range(1, N_DEV):
            rdma = pltpu.make_async_remote_copy(
                src_ref=kvfull.at[my],
                dst_ref=kvfull.at[my],
                send_sem=send_sems.at[d - 1],
                recv_sem=recv_sems.at[d - 1],
                device_id=(lax.rem(my + d, N_DEV),),
                device_id_type=pl.DeviceIdType.MESH,
            )
            rdma.start()
            sends.append(rdma)

        sblk = [[[] for _ in range(HQ)] for _ in range(B)]
        vblk = [[[] for _ in range(HQ)] for _ in range(B)]

        def consume(origin):
            kv_b = [kvfull[origin, b] for b in range(B)]
            for b in range(B):
                ko = kv_b[b][0:SQ_LOCAL]
                vo = kv_b[b][SQ_LOCAL:2 * SQ_LOCAL]
                for hh in range(HQ):
                    qh = qs[b][:, hh * DH:(hh + 1) * DH]
                    s = lax.dot_general(
                        qh, ko[:, hh * DH:(hh + 1) * DH],
                        (((1,), (1,)), ((), ())), preferred_element_type=f32)
                    sblk[b][hh].append(s)
                    vblk[b][hh].append(vo[:, hh * DH:(hh + 1) * DH])

        consume(my)

        for d in range(1, N_DEV):
            origin = lax.rem(my - d + N_DEV, N_DEV)
            recv = pltpu.make_async_remote_copy(
                src_ref=kvfull.at[my],
                dst_ref=kvfull.at[origin],
                send_sem=send_sems.at[d - 1],
                recv_sem=recv_sems.at[d - 1],
                device_id=(my,),
                device_id_type=pl.DeviceIdType.MESH,
            )
            recv.wait_recv()
            consume(origin)

        for b in range(B):
            ctx_heads = []
            for hh in range(HQ):
                s = jnp.concatenate(sblk[b][hh], axis=1) * f32(0.125)
                m = jnp.max(s, axis=1, keepdims=True)
                w = jnp.exp(s - m)
                w = w / jnp.sum(w, axis=1, keepdims=True)
                ctx = jnp.zeros((SQ_LOCAL, DH), f32)
                for j in range(N_DEV):
                    wj = w[:, j * SQ_LOCAL:(j + 1) * SQ_LOCAL]
                    ctx = ctx + jnp.dot(wj, vblk[b][hh][j],
                                        preferred_element_type=f32)
                ctx_heads.append(ctx)
            ctx_b = jnp.concatenate(ctx_heads, axis=1)
            out_ref[b] = jnp.dot(ctx_b, wo_ref[...], preferred_element_type=f32)

        for rdma in sends:
            rdma.wait_send()

    return pl.pallas_call(
        body,
        out_shape=jax.ShapeDtypeStruct((B, SQ_LOCAL, D), jnp.float32),
        in_specs=[pl.BlockSpec(memory_space=pltpu.VMEM)] * 5,
        out_specs=pl.BlockSpec(memory_space=pltpu.VMEM),
        scratch_shapes=[
            pltpu.VMEM((N_DEV, B, 2 * SQ_LOCAL, HD), jnp.float32),
            pltpu.SemaphoreType.DMA((N_DEV - 1,)),
            pltpu.SemaphoreType.DMA((N_DEV - 1,)),
        ],
        compiler_params=pltpu.CompilerParams(collective_id=0),
    )(x, Wq, Wk, Wv, Wo)


# device time: 30752 ns/iter; 2.2269x vs baseline; 1.5632x over previous
import jax
import jax.numpy as jnp
import numpy as np
from jax import lax
from jax.experimental import pallas as pl
from jax.experimental.pallas import tpu as pltpu

N_DEV = 8
B = 2
SQ_LOCAL = 128
D = 512
HQ = 4
DH = 64
HD = HQ * DH


def kernel(x, Wq, Wk, Wv, Wo):
    def body(x_ref, wq_ref, wk_ref, wv_ref, wo_ref, out_ref,
             kvfull, send_sems, recv_sems):
        f32 = jnp.float32
        bf16 = jnp.bfloat16
        my = lax.axis_index("i")

        barrier_sem = pltpu.get_barrier_semaphore()
        for d in range(1, N_DEV):
            pl.semaphore_signal(
                barrier_sem, inc=1,
                device_id=(lax.rem(my + d, N_DEV),),
                device_id_type=pl.DeviceIdType.MESH,
            )
        pl.semaphore_wait(barrier_sem, N_DEV - 1)

        lane = lax.broadcasted_iota(jnp.int32, (SQ_LOCAL, HD), 1)
        d_in_head = lax.rem(lane, DH)
        pair = (d_in_head // 2) * 2
        freq = jnp.exp(pair.astype(f32) * f32(-np.log(10000.0) / DH))
        pos = (my * SQ_LOCAL
               + lax.broadcasted_iota(jnp.int32, (SQ_LOCAL, HD), 0)).astype(f32)
        ang = pos * freq
        cos = jnp.cos(ang)
        sin = jnp.sin(ang)

        kk = lax.broadcasted_iota(jnp.int32, (HD, HD), 0)
        jj = lax.broadcasted_iota(jnp.int32, (HD, HD), 1)
        rot_pos = ((lax.rem(jj, 2) == 1) & (kk == jj - 1)).astype(bf16)
        rot_neg = ((lax.rem(jj, 2) == 0) & (kk == jj + 1)).astype(bf16)
        rmat = rot_pos - rot_neg

        wq16 = wq_ref[...].astype(bf16)
        wk16 = wk_ref[...].astype(bf16)
        wv16 = wv_ref[...].astype(bf16)
        qs = []
        for b in range(B):
            xb = x_ref[b].astype(bf16)
            q = jnp.dot(xb, wq16, preferred_element_type=f32)
            k = jnp.dot(xb, wk16, preferred_element_type=f32)
            v = jnp.dot(xb, wv16, preferred_element_type=f32)
            q16 = q.astype(bf16)
            k16 = k.astype(bf16)
            qrot = jnp.dot(q16, rmat, preferred_element_type=f32)
            krot = jnp.dot(k16, rmat, preferred_element_type=f32)
            q = q * cos + qrot * sin
            k = k * cos + krot * sin
            qs.append(q.astype(bf16))
            kvfull[my, b, 0:SQ_LOCAL] = k.astype(bf16)
            kvfull[my, b, SQ_LOCAL:2 * SQ_LOCAL] = v.astype(bf16)

        sends = []
        for d in range(1, N_DEV):
            rdma = pltpu.make_async_remote_copy(
                src_ref=kvfull.at[my],
                dst_ref=kvfull.at[my],
                send_sem=send_sems.at[d - 1],
                recv_sem=recv_sems.at[d - 1],
                device_id=(lax.rem(my + d, N_DEV),),
                device_id_type=pl.DeviceIdType.MESH,
            )
            rdma.start()
            sends.append(rdma)

        sblk = [[[] for _ in range(HQ)] for _ in range(B)]
        vblk = [[[] for _ in range(HQ)] for _ in range(B)]

        def consume(origin):
            kv_b = [kvfull[origin, b] for b in range(B)]
            for b in range(B):
                ko = kv_b[b][0:SQ_LOCAL]
                vo = kv_b[b][SQ_LOCAL:2 * SQ_LOCAL]
                for hh in range(HQ):
                    qh = qs[b][:, hh * DH:(hh + 1) * DH]
                    s = lax.dot_general(
                        qh, ko[:, hh * DH:(hh + 1) * DH],
                        (((1,), (1,)), ((), ())), preferred_element_type=f32)
                    sblk[b][hh].append(s)
                    vblk[b][hh].append(vo[:, hh * DH:(hh + 1) * DH])

        consume(my)

        for d in range(1, N_DEV):
            origin = lax.rem(my - d + N_DEV, N_DEV)
            recv = pltpu.make_async_remote_copy(
                src_ref=kvfull.at[my],
                dst_ref=kvfull.at[origin],
                send_sem=send_sems.at[d - 1],
                recv_sem=recv_sems.at[d - 1],
                device_id=(my,),
                device_id_type=pl.DeviceIdType.MESH,
            )
            recv.wait_recv()
            consume(origin)

        wo16 = wo_ref[...].astype(bf16)
        for b in range(B):
            ctx_heads = []
            for hh in range(HQ):
                s = jnp.concatenate(sblk[b][hh], axis=1) * f32(0.125)
                m = jnp.max(s, axis=1, keepdims=True)
                w = jnp.exp(s - m)
                w = (w / jnp.sum(w, axis=1, keepdims=True)).astype(bf16)
                ctx = jnp.zeros((SQ_LOCAL, DH), f32)
                for j in range(N_DEV):
                    wj = w[:, j * SQ_LOCAL:(j + 1) * SQ_LOCAL]
                    ctx = ctx + jnp.dot(wj, vblk[b][hh][j],
                                        preferred_element_type=f32)
                ctx_heads.append(ctx.astype(bf16))
            ctx_b = jnp.concatenate(ctx_heads, axis=1)
            out_ref[b] = jnp.dot(ctx_b, wo16, preferred_element_type=f32)

        for rdma in sends:
            rdma.wait_send()

    return pl.pallas_call(
        body,
        out_shape=jax.ShapeDtypeStruct((B, SQ_LOCAL, D), jnp.float32),
        in_specs=[pl.BlockSpec(memory_space=pltpu.VMEM)] * 5,
        out_specs=pl.BlockSpec(memory_space=pltpu.VMEM),
        scratch_shapes=[
            pltpu.VMEM((N_DEV, B, 2 * SQ_LOCAL, HD), jnp.bfloat16),
            pltpu.SemaphoreType.DMA((N_DEV - 1,)),
            pltpu.SemaphoreType.DMA((N_DEV - 1,)),
        ],
        compiler_params=pltpu.CompilerParams(collective_id=0),
    )(x, Wq, Wk, Wv, Wo)
